# baseline (speedup 1.0000x reference)
import functools

import jax
import jax.numpy as jnp
from jax.experimental import pallas as pl
from jax.experimental.pallas import tpu as pltpu


def _fused_kernel(x_ref, a_ref, bg_ref, wg_ref, wd_ref, bd_ref, out_hbm,
                  g_ref, stage_ref, sems, *, NC, NK, BW, B, H):
    s = pl.program_id(0)

    def out_copy(w, b):
        return pltpu.make_async_copy(
            stage_ref.at[b % 2],
            out_hbm.at[b, pl.ds(w * BW, BW), :],
            sems.at[b % 2],
        )

    for k in range(NC):
        @pl.when(s == k)
        def _read_acc(k=k):
            wg = wg_ref[...].astype(jnp.bfloat16)
            hs = [
                jax.lax.dot_general(
                    x_ref[b].astype(jnp.bfloat16), wg,
                    dimension_numbers=(((0,), (0,)), ((), ())),
                    preferred_element_type=jnp.float32,
                )
                for b in range(B)
            ]
            panel = jnp.concatenate(hs, axis=1).astype(jnp.bfloat16)
            acc = jnp.dot(a_ref[...].astype(jnp.bfloat16), panel,
                          preferred_element_type=jnp.float32)
            if k == 0:
                g_ref[...] = acc
            else:
                g_ref[...] += acc

    for w in range(NC):
        @pl.when(s == NC + w)
        def _write(w=w):
            g = g_ref[pl.ds(w * BW, BW), :]
            wd = wd_ref[...].astype(jnp.bfloat16)
            for b in range(B):
                gb = jnp.maximum(g[:, b * H:(b + 1) * H] + bg_ref[...], 0.0)
                ob = jnp.dot(gb.astype(jnp.bfloat16), wd,
                             preferred_element_type=jnp.float32)
                if b >= 2:
                    out_copy(w, b - 2).wait()
                stage_ref[b % 2] = jnp.maximum(ob + bd_ref[...], 0.0)
                out_copy(w, b).start()
            out_copy(w, B - 2).wait()
            out_copy(w, B - 1).wait()


def kernel(x, a, W_gcn, b_gcn, W_d, b_d):
    B, T, N = x.shape
    H = W_gcn.shape[1]
    bg = b_gcn.reshape(1, H)
    bd = b_d.reshape(1, T)

    NC = 4
    NK = N // NC
    BW = N // NC

    return pl.pallas_call(
        functools.partial(_fused_kernel, NC=NC, NK=NK, BW=BW, B=B, H=H),
        grid=(2 * NC,),
        in_specs=[
            pl.BlockSpec((B, T, NK), lambda s: (0, 0, jnp.minimum(s, NC - 1))),
            pl.BlockSpec((N, NK), lambda s: (0, jnp.minimum(s, NC - 1))),
            pl.BlockSpec((1, H), lambda s: (0, 0)),
            pl.BlockSpec((T, H), lambda s: (0, 0)),
            pl.BlockSpec((H, T), lambda s: (0, 0)),
            pl.BlockSpec((1, T), lambda s: (0, 0)),
        ],
        out_specs=pl.BlockSpec(memory_space=pl.ANY),
        out_shape=jax.ShapeDtypeStruct((B, N, T), jnp.float32),
        scratch_shapes=[
            pltpu.VMEM((N, B * H), jnp.float32),
            pltpu.VMEM((2, BW, T), jnp.float32),
            pltpu.SemaphoreType.DMA((2,)),
        ],
    )(x, a, bg, W_gcn, W_d, bd)


# speedup vs baseline: 1.6782x; 1.6782x over previous
import functools

import jax
import jax.numpy as jnp
from jax.experimental import pallas as pl
from jax.experimental.pallas import tpu as pltpu


def _fused_kernel(x_ref, a_ref, bg_ref, wg_ref, wd_ref, bd_ref, out_ref,
                  g_ref, *, NC, NCW, NK, BW, B, H):
    s = pl.program_id(0)

    for k in range(NC):
        @pl.when(s == k)
        def _read_acc(k=k):
            wg = wg_ref[...].astype(jnp.bfloat16)
            hs = [
                jax.lax.dot_general(
                    x_ref[b].astype(jnp.bfloat16), wg,
                    dimension_numbers=(((0,), (0,)), ((), ())),
                    preferred_element_type=jnp.float32,
                )
                for b in range(B)
            ]
            panel = jnp.concatenate(hs, axis=1).astype(jnp.bfloat16)
            acc = jnp.dot(a_ref[...].astype(jnp.bfloat16), panel,
                          preferred_element_type=jnp.float32)
            if k == 0:
                g_ref[...] = acc
            else:
                g_ref[...] += acc

    for w in range(NCW):
        @pl.when(s == NC + w)
        def _write(w=w):
            g = g_ref[pl.ds(w * BW, BW), :]
            wd = wd_ref[...].astype(jnp.bfloat16)
            for b in range(B):
                gb = jnp.maximum(g[:, b * H:(b + 1) * H] + bg_ref[...], 0.0)
                ob = jnp.dot(gb.astype(jnp.bfloat16), wd,
                             preferred_element_type=jnp.float32)
                out_ref[b] = jnp.maximum(ob + bd_ref[...], 0.0)


def kernel(x, a, W_gcn, b_gcn, W_d, b_d):
    B, T, N = x.shape
    H = W_gcn.shape[1]
    bg = b_gcn.reshape(1, H)
    bd = b_d.reshape(1, T)

    NC = 4
    NCW = 8
    NK = N // NC
    BW = N // NCW

    return pl.pallas_call(
        functools.partial(_fused_kernel, NC=NC, NCW=NCW, NK=NK, BW=BW, B=B, H=H),
        grid=(NC + NCW,),
        in_specs=[
            pl.BlockSpec((B, T, NK), lambda s: (0, 0, jnp.minimum(s, NC - 1))),
            pl.BlockSpec((N, NK), lambda s: (0, jnp.minimum(s, NC - 1))),
            pl.BlockSpec((1, H), lambda s: (0, 0)),
            pl.BlockSpec((T, H), lambda s: (0, 0)),
            pl.BlockSpec((H, T), lambda s: (0, 0)),
            pl.BlockSpec((1, T), lambda s: (0, 0)),
        ],
        out_specs=pl.BlockSpec(
            (B, BW, T), lambda s: (0, jnp.maximum(s - NC, 0), 0)),
        out_shape=jax.ShapeDtypeStruct((B, N, T), jnp.float32),
        scratch_shapes=[pltpu.VMEM((N, B * H), jnp.float32)],
    )(x, a, bg, W_gcn, W_d, bd)


# speedup vs baseline: 1.7231x; 1.0267x over previous
import functools

import jax
import jax.numpy as jnp
from jax.experimental import pallas as pl
from jax.experimental.pallas import tpu as pltpu


def _fused_kernel(x_ref, a_ref, bg_ref, wg_ref, wd_ref, bd_ref, out_ref,
                  g_ref, *, NC, NK, BW, B, H):
    s = pl.program_id(0)

    for k in range(NC):
        @pl.when(s == k)
        def _read_acc(k=k):
            wg = wg_ref[...].astype(jnp.bfloat16)
            hs = [
                jax.lax.dot_general(
                    x_ref[b].astype(jnp.bfloat16), wg,
                    dimension_numbers=(((0,), (0,)), ((), ())),
                    preferred_element_type=jnp.float32,
                )
                for b in range(B)
            ]
            panel = jnp.concatenate(hs, axis=1).astype(jnp.bfloat16)
            acc = jnp.dot(a_ref[...].astype(jnp.bfloat16), panel,
                          preferred_element_type=jnp.float32)
            if k == 0:
                g_ref[...] = acc
            else:
                g_ref[...] += acc

    for w in range(NC):
        @pl.when(s == NC + w)
        def _write(w=w):
            g = g_ref[pl.ds(w * BW, BW), :]
            wd = wd_ref[...].astype(jnp.bfloat16)
            for b in range(B):
                gb = jnp.maximum(g[:, b * H:(b + 1) * H] + bg_ref[...], 0.0)
                ob = jnp.dot(gb.astype(jnp.bfloat16), wd,
                             preferred_element_type=jnp.float32)
                out_ref[b] = jnp.maximum(ob + bd_ref[...], 0.0)


def kernel(x, a, W_gcn, b_gcn, W_d, b_d):
    B, T, N = x.shape
    H = W_gcn.shape[1]
    bg = b_gcn.reshape(1, H)
    bd = b_d.reshape(1, T)

    NC = 4
    NK = N // NC
    BW = N // NC

    return pl.pallas_call(
        functools.partial(_fused_kernel, NC=NC, NK=NK, BW=BW, B=B, H=H),
        grid=(2 * NC,),
        in_specs=[
            pl.BlockSpec((B, T, NK), lambda s: (0, 0, jnp.minimum(s, NC - 1))),
            pl.BlockSpec((N, NK), lambda s: (0, jnp.minimum(s, NC - 1))),
            pl.BlockSpec((1, H), lambda s: (0, 0)),
            pl.BlockSpec((T, H), lambda s: (0, 0)),
            pl.BlockSpec((H, T), lambda s: (0, 0)),
            pl.BlockSpec((1, T), lambda s: (0, 0)),
        ],
        out_specs=pl.BlockSpec(
            (B, BW, T), lambda s: (0, jnp.maximum(s - NC, 0), 0)),
        out_shape=jax.ShapeDtypeStruct((B, N, T), jnp.float32),
        scratch_shapes=[pltpu.VMEM((N, B * H), jnp.float32)],
    )(x, a, bg, W_gcn, W_d, bd)
